# baseline (speedup 1.0000x reference)
import functools

import jax
import jax.numpy as jnp
from jax import lax
from jax.experimental import pallas as pl
from jax.experimental.pallas import tpu as pltpu
from jax.experimental.pallas import tpu_sc as plsc

_B = 16
_P = 4096
_S = 256
_L = 16
_NS = 16
_WPT = _B * _P * _S // 32
_CW = 32768
_NCHUNK = _WPT // _CW
_GROUPS = _CW // (_L * _S)

_sc_mesh = plsc.VectorSubcoreMesh(core_axis_name="c", subcore_axis_name="s")


@functools.partial(
    pl.kernel,
    mesh=_sc_mesh,
    compiler_params=pltpu.CompilerParams(needs_layout_passes=False),
    out_type=jax.ShapeDtypeStruct((_B, _S), jnp.float32),
    scratch_types=[
        pltpu.VMEM((_CW,), jnp.int32),
        pltpu.VMEM((_CW,), jnp.int32),
        pltpu.VMEM((_CW // _S * _L,), jnp.int32),
        pltpu.VMEM((_L * _S,), jnp.int32),
        pltpu.VMEM((_S,), jnp.int32),
        pltpu.VMEM((_S,), jnp.int32),
        pltpu.VMEM((_S,), jnp.float32),
        pltpu.VMEM_SHARED((_NS, _S), jnp.int32),
        pltpu.SemaphoreType.DMA,
        pltpu.SemaphoreType.DMA,
    ],
)
def _probs_sc(x_hbm, out_hbm, buf0, buf1, colsums, h2d, hrow, prt, prow, shared, s0, s1):
    c = lax.axis_index("c")
    s = lax.axis_index("s")
    wid = c * _NS + s
    base = wid * _WPT

    iota = lax.iota(jnp.int32, _L)
    lane_off = iota * _S
    ones = jnp.ones((_L,), jnp.int32)
    zeros = jnp.zeros((_L,), jnp.int32)
    perms = [jnp.bitwise_and(iota + j, _L - 1) for j in range(_L)]

    def zbody(j, carry):
        h2d[pl.ds(j * _L, _L)] = zeros
        return carry

    lax.fori_loop(0, (_L * _S) // _L, zbody, 0)

    def start(cidx, buf, sem):
        pltpu.async_copy(x_hbm.at[pl.ds(base + cidx * _CW, _CW)], buf, sem)

    def wait(buf, sem):
        pltpu.make_async_copy(x_hbm.at[pl.ds(0, _CW)], buf, sem).wait()

    tpose = iota * _L

    def process(buf):
        def pbody(p):
            pb = p * _S
            vs = [buf[pl.ds(pb + r * _L, _L)] for r in range(_L)]
            while len(vs) > 1:
                vs = [vs[i] + vs[i + 1] for i in range(0, len(vs), 2)]
            colsums[pl.ds(p * _L, _L)] = vs[0]

        plsc.parallel_loop(0, _CW // _S, 1, unroll=2)(pbody)

        def gbody(g, carry):
            gb = tpose + g * (_L * _L)
            acc = zeros
            for j in range(_L):
                acc = acc + plsc.load_gather(colsums, [gb + perms[j]])
            plsc.addupdate_scatter(h2d, [acc + lane_off], ones, mask=acc < _S)
            return carry

        lax.fori_loop(0, _GROUPS, gbody, 0)

    start(0, buf0, s0)
    start(1, buf1, s1)

    def cbody(cpair, carry):
        cidx = cpair * 2
        wait(buf0, s0)
        process(buf0)

        @pl.when(cidx + 2 < _NCHUNK)
        def _():
            start(cidx + 2, buf0, s0)

        wait(buf1, s1)
        process(buf1)

        @pl.when(cidx + 3 < _NCHUNK)
        def _():
            start(cidx + 3, buf1, s1)

        return carry

    lax.fori_loop(0, _NCHUNK // 2, cbody, 0)

    def rbody(j, carry):
        acc = h2d[pl.ds(j * _L, _L)]
        for l in range(1, _L):
            acc = acc + h2d[pl.ds(l * _S + j * _L, _L)]
        hrow[pl.ds(j * _L, _L)] = acc
        return carry

    lax.fori_loop(0, _S // _L, rbody, 0)

    pltpu.sync_copy(hrow, shared.at[s])
    plsc.subcore_barrier()

    @pl.when(lax.rem(s, 2) == 0)
    def _():
        pltpu.sync_copy(shared.at[s + 1], prt)

        def mbody(j, tot):
            v = hrow[pl.ds(j * _L, _L)] + prt[pl.ds(j * _L, _L)]
            vf = v.astype(jnp.float32)
            prow[pl.ds(j * _L, _L)] = vf
            return tot + vf

        tot_vec = lax.fori_loop(0, _S // _L, mbody, jnp.zeros((_L,), jnp.float32))
        total = lax.broadcast_in_dim(jnp.sum(tot_vec), (_L,), ())

        def nbody(j, carry):
            prow[pl.ds(j * _L, _L)] = prow[pl.ds(j * _L, _L)] / total
            return carry

        lax.fori_loop(0, _S // _L, nbody, 0)
        row = c * (_NS // 2) + lax.div(s, 2)
        pltpu.sync_copy(prow, out_hbm.at[row])


def kernel(inputs):
    x = inputs.reshape(_B * _P * _S)
    probs = _probs_sc(x)
    return ((probs,),)


# speedup vs baseline: 4.0124x; 4.0124x over previous
import functools

import jax
import jax.numpy as jnp
from jax import lax
from jax.experimental import pallas as pl
from jax.experimental.pallas import tpu as pltpu
from jax.experimental.pallas import tpu_sc as plsc

_B = 16
_P = 4096
_S = 256
_L = 16
_NS = 16
_WPT = _B * _P * _S // 32
_CW = 32768
_NCHUNK = _WPT // _CW
_GROUPS = _CW // (_L * _S)

_sc_mesh = plsc.VectorSubcoreMesh(core_axis_name="c", subcore_axis_name="s")


@functools.partial(
    pl.kernel,
    mesh=_sc_mesh,
    compiler_params=pltpu.CompilerParams(needs_layout_passes=False),
    out_type=jax.ShapeDtypeStruct((_B, _S), jnp.float32),
    scratch_types=[
        pltpu.VMEM((_CW // _S, _S), jnp.int32),
        pltpu.VMEM((_CW // _S, _S), jnp.int32),
        pltpu.VMEM((_CW // _S * _L,), jnp.int32),
        pltpu.VMEM((_L * _S,), jnp.int32),
        pltpu.VMEM((_S,), jnp.int32),
        pltpu.VMEM((_S,), jnp.int32),
        pltpu.VMEM((_S,), jnp.float32),
        pltpu.VMEM_SHARED((_NS, _S), jnp.int32),
        pltpu.SemaphoreType.DMA,
        pltpu.SemaphoreType.DMA,
    ],
)
def _probs_sc(x_hbm, out_hbm, buf0, buf1, colsums, h2d, hrow, prt, prow, shared, s0, s1):
    c = lax.axis_index("c")
    s = lax.axis_index("s")
    wid = c * _NS + s
    row = lax.div(wid, 2)
    half = lax.rem(wid, 2)
    cpp = _CW // _S

    iota = lax.iota(jnp.int32, _L)
    lane_off = iota * _S
    ones = jnp.ones((_L,), jnp.int32)
    zeros = jnp.zeros((_L,), jnp.int32)
    perms = [jnp.bitwise_and(iota + j, _L - 1) for j in range(_L)]

    def zbody(j, carry):
        h2d[pl.ds(j * _L, _L)] = zeros
        return carry

    lax.fori_loop(0, (_L * _S) // _L, zbody, 0)

    def start(cidx, buf, sem):
        pstart = half * (_P // 2) + cidx * cpp
        pltpu.async_copy(x_hbm.at[row, pl.ds(pstart, cpp)], buf, sem)

    def wait(buf, sem):
        pltpu.make_async_copy(x_hbm.at[0, pl.ds(0, cpp)], buf, sem).wait()

    tpose = iota * _L

    def process(buf):
        def pbody(p):
            vs = [buf[p, pl.ds(r * _L, _L)] for r in range(_L)]
            while len(vs) > 1:
                vs = [vs[i] + vs[i + 1] for i in range(0, len(vs), 2)]
            colsums[pl.ds(p * _L, _L)] = vs[0]

        plsc.parallel_loop(0, _CW // _S, 1, unroll=2)(pbody)

        def gbody(g, carry):
            gb = tpose + g * (_L * _L)
            acc = zeros
            for j in range(_L):
                acc = acc + plsc.load_gather(colsums, [gb + perms[j]])
            plsc.addupdate_scatter(h2d, [acc + lane_off], ones, mask=acc < _S)
            return carry

        lax.fori_loop(0, _GROUPS, gbody, 0)

    start(0, buf0, s0)
    start(1, buf1, s1)

    def cbody(cpair, carry):
        cidx = cpair * 2
        wait(buf0, s0)
        process(buf0)

        @pl.when(cidx + 2 < _NCHUNK)
        def _():
            start(cidx + 2, buf0, s0)

        wait(buf1, s1)
        process(buf1)

        @pl.when(cidx + 3 < _NCHUNK)
        def _():
            start(cidx + 3, buf1, s1)

        return carry

    lax.fori_loop(0, _NCHUNK // 2, cbody, 0)

    def rbody(j, carry):
        acc = h2d[pl.ds(j * _L, _L)]
        for l in range(1, _L):
            acc = acc + h2d[pl.ds(l * _S + j * _L, _L)]
        hrow[pl.ds(j * _L, _L)] = acc
        return carry

    lax.fori_loop(0, _S // _L, rbody, 0)

    pltpu.sync_copy(hrow, shared.at[s])
    plsc.subcore_barrier()

    @pl.when(lax.rem(s, 2) == 0)
    def _():
        pltpu.sync_copy(shared.at[s + 1], prt)

        def mbody(j, tot):
            v = hrow[pl.ds(j * _L, _L)] + prt[pl.ds(j * _L, _L)]
            vf = v.astype(jnp.float32)
            prow[pl.ds(j * _L, _L)] = vf
            return tot + vf

        tot_vec = lax.fori_loop(0, _S // _L, mbody, jnp.zeros((_L,), jnp.float32))
        total = lax.broadcast_in_dim(jnp.sum(tot_vec), (_L,), ())

        def nbody(j, carry):
            prow[pl.ds(j * _L, _L)] = prow[pl.ds(j * _L, _L)] / total
            return carry

        lax.fori_loop(0, _S // _L, nbody, 0)
        pltpu.sync_copy(prow, out_hbm.at[row])


def kernel(inputs):
    x = inputs.reshape(_B, _P, _S)
    probs = _probs_sc(x)
    return ((probs,),)


# speedup vs baseline: 7.6925x; 1.9172x over previous
import functools

import jax
import jax.numpy as jnp
from jax import lax
from jax.experimental import pallas as pl
from jax.experimental.pallas import tpu as pltpu
from jax.experimental.pallas import tpu_sc as plsc

_B = 16
_P = 4096
_S = 256
_L = 16
_NS = 16
_NPT = 32
_NCHUNK = 16

_sc_mesh = plsc.VectorSubcoreMesh(core_axis_name="c", subcore_axis_name="s")


@functools.partial(
    pl.kernel,
    mesh=_sc_mesh,
    compiler_params=pltpu.CompilerParams(needs_layout_passes=False),
    out_type=jax.ShapeDtypeStruct((_B, _S), jnp.float32),
    scratch_types=[
        pltpu.VMEM((_NPT, 8, 128), jnp.int32),
        pltpu.VMEM((_NPT, 8, 128), jnp.int32),
        pltpu.VMEM((_P,), jnp.int32),
        pltpu.VMEM((_P,), jnp.int32),
        pltpu.VMEM((_L * _S,), jnp.int32),
        pltpu.VMEM((_S,), jnp.float32),
        pltpu.VMEM_SHARED((_NS, _P), jnp.int32),
        pltpu.SemaphoreType.DMA,
        pltpu.SemaphoreType.DMA,
    ],
)
def _probs_sc(x_hbm, out_hbm, buf0, buf1, acc, prt, h2d, prow, shared, s0, s1):
    c = lax.axis_index("c")
    s = lax.axis_index("s")
    wid = c * _NS + s
    row = lax.div(wid, 2)
    half = lax.rem(wid, 2)

    iota = lax.iota(jnp.int32, _L)
    lane_off = iota * _S
    ones = jnp.ones((_L,), jnp.int32)
    zeros = jnp.zeros((_L,), jnp.int32)

    def zbody(j, carry):
        acc[pl.ds(j * _L, _L)] = zeros
        return carry

    lax.fori_loop(0, _P // _L, zbody, 0)

    def zhbody(j, carry):
        h2d[pl.ds(j * _L, _L)] = zeros
        return carry

    lax.fori_loop(0, (_L * _S) // _L, zhbody, 0)

    def start(t, buf, sem):
        r = half * 8 + lax.div(t, 2)
        ct = lax.rem(t, 2)
        pltpu.async_copy(x_hbm.at[row, r, ct], buf, sem)

    def wait(buf, sem):
        pltpu.make_async_copy(x_hbm.at[0, 0, 0], buf, sem).wait()

    def process(buf):
        def ptbody(pt):
            for ch in range(8):
                vs = [buf[pt, r8, pl.ds(ch * _L, _L)] for r8 in range(8)]
                while len(vs) > 1:
                    vs = [vs[i] + vs[i + 1] for i in range(0, len(vs), 2)]
                a = pl.ds(pt * 128 + ch * _L, _L)
                acc[a] = acc[a] + vs[0]

        plsc.parallel_loop(0, _NPT, 1, unroll=2)(ptbody)

    start(0, buf0, s0)
    start(1, buf1, s1)

    def cbody(cpair, carry):
        t = cpair * 2
        wait(buf0, s0)
        process(buf0)

        @pl.when(t + 2 < _NCHUNK)
        def _():
            start(t + 2, buf0, s0)

        wait(buf1, s1)
        process(buf1)

        @pl.when(t + 3 < _NCHUNK)
        def _():
            start(t + 3, buf1, s1)

        return carry

    lax.fori_loop(0, _NCHUNK // 2, cbody, 0)

    pltpu.sync_copy(acc, shared.at[s])
    plsc.subcore_barrier()

    @pl.when(lax.rem(s, 2) == 0)
    def _():
        pltpu.sync_copy(shared.at[s + 1], prt)

        def hbody(j, carry):
            cnt = acc[pl.ds(j * _L, _L)] + prt[pl.ds(j * _L, _L)]
            plsc.addupdate_scatter(h2d, [cnt + lane_off], ones, mask=cnt < _S)
            return carry

        lax.fori_loop(0, _P // _L, hbody, 0)

        def mbody(j, tot):
            hv = h2d[pl.ds(j * _L, _L)]
            for l in range(1, _L):
                hv = hv + h2d[pl.ds(l * _S + j * _L, _L)]
            vf = hv.astype(jnp.float32)
            prow[pl.ds(j * _L, _L)] = vf
            return tot + vf

        tot_vec = lax.fori_loop(0, _S // _L, mbody, jnp.zeros((_L,), jnp.float32))
        total = lax.broadcast_in_dim(jnp.sum(tot_vec), (_L,), ())

        def nbody(j, carry):
            prow[pl.ds(j * _L, _L)] = prow[pl.ds(j * _L, _L)] / total
            return carry

        lax.fori_loop(0, _S // _L, nbody, 0)
        pltpu.sync_copy(prow, out_hbm.at[row])


def kernel(inputs):
    x = inputs.reshape(_B, _P, _L, _L)
    x = x.transpose(0, 2, 3, 1)
    x = x.reshape(_B, _L, 2, 8, _NPT, 128)
    x = x.transpose(0, 1, 2, 4, 3, 5)
    probs = _probs_sc(x)
    return ((probs,),)
